# baseline (speedup 1.0000x reference)
import jax
import jax.numpy as jnp
from jax import lax
from jax.experimental import pallas as pl
from jax.experimental.pallas import tpu as pltpu
from jax.experimental.pallas import tpu_sc as plsc

_NC = 2
_NS = 16
_NW = _NC * _NS
_L = 16


def _make_scatter_add(M, D, B):
  mesh = plsc.VectorSubcoreMesh(
      core_axis_name="c", subcore_axis_name="s",
      num_cores=_NC, num_subcores=_NS)
  rpw = (M + _NW - 1) // _NW
  nvec = B // _L
  nchunk = D // _L
  KB = _L
  rpw_pad = ((rpw + KB - 1) // KB) * KB

  def body(out_hbm, idx_hbm, msk_hbm, upd_hbm,
           idx_v, msk_v, mi_v, mb_v, oi_v, ob_v, tab_v,
           acc_v, upd_v, vi_s, vb_s, acc_o, upd_o, vi_o, vb_o, gsem, ssem):
    c = lax.axis_index("c")
    s = lax.axis_index("s")
    wid = s * _NC + c
    lo = wid * rpw
    hi = jnp.minimum(lo + rpw, M)
    lanes = lax.iota(jnp.int32, _L)
    zeros = jnp.zeros((_L,), jnp.int32)
    ones = zeros + 1

    pltpu.sync_copy(idx_hbm, idx_v)
    pltpu.sync_copy(msk_hbm, msk_v)

    def clr_body(t, a):
      tab_v[pl.ds(t * _L, _L)] = zeros
      return a

    lax.fori_loop(0, rpw_pad // _L, clr_body, jnp.int32(0))

    def scan_body(j, cnt):
      base = j * _L
      vi = idx_v[pl.ds(base, _L)]
      vm = msk_v[pl.ds(base, _L)]
      m = (vi >= lo) & (vi < hi) & (vm != 0)
      rel = jnp.where(m, vi - lo, 0)
      c1 = plsc.cumsum(jnp.where(m, ones, zeros))
      pos = (cnt + c1) - 1
      plsc.store_scatter(tab_v, [rel], pos + 1, mask=m)
      plsc.store_scatter(idx_v, [pos], vi, mask=m)
      plsc.store_scatter(msk_v, [pos], base + lanes, mask=m)
      return cnt + jnp.sum(jnp.where(m, ones, zeros))

    cnt = lax.fori_loop(0, nvec, scan_body, jnp.int32(0))

    nl = (cnt + (_L - 1)) // _L

    def part_body(j, carry):
      cntm, cnto = carry
      base = j * _L
      pvec = base + lanes
      valid = pvec < cnt
      vi = idx_v[pl.ds(base, _L)]
      vb = msk_v[pl.ds(base, _L)]
      rel = jnp.where(valid, vi - lo, 0)
      tv = plsc.load_gather(tab_v, [rel])
      first = valid & (tv == pvec + 1)
      dup = valid & ~first
      c1 = plsc.cumsum(jnp.where(first, ones, zeros))
      posm = (cntm + c1) - 1
      plsc.store_scatter(mi_v, [posm], vi, mask=first)
      plsc.store_scatter(mb_v, [posm], vb, mask=first)
      c2 = plsc.cumsum(jnp.where(dup, ones, zeros))
      poso = (cnto + c2) - 1
      plsc.store_scatter(oi_v, [poso], vi, mask=dup)
      plsc.store_scatter(ob_v, [poso], vb, mask=dup)
      return (cntm + jnp.sum(jnp.where(first, ones, zeros)),
              cnto + jnp.sum(jnp.where(dup, ones, zeros)))

    cntm, cnto = lax.fori_loop(0, nl, part_body,
                               (jnp.int32(0), jnp.int32(0)))
    nbm = (cntm + (KB - 1)) // KB

    def add_rows(acc_r, upd_r, nrows):
      def add_body(i, a):
        rr = i // nchunk
        jj = i % nchunk
        u = upd_r[rr, pl.ds(jj * _L, _L)]
        plsc.addupdate(acc_r.at[rr, pl.ds(jj * _L, _L)], u)
        return a

      lax.fori_loop(0, nrows * nchunk, add_body, jnp.int32(0), unroll=16)

    def main_body(j, carry):
      base = j * KB
      b16 = jnp.broadcast_to(base, (_L,))
      vi0 = plsc.load_gather(mi_v, [b16])
      vb0 = plsc.load_gather(mb_v, [b16])
      for h in range(KB // _L):
        hb = base + h * _L
        valid = (hb + lanes) < cntm
        vi = mi_v[pl.ds(hb, _L)]
        vb = mb_v[pl.ds(hb, _L)]
        vi_s[pl.ds(h * _L, _L)] = jnp.where(valid, vi, vi0)
        vb_s[pl.ds(h * _L, _L)] = jnp.where(valid, vb, vb0)
      pltpu.async_copy(out_hbm.at[vi_s], acc_v, gsem)
      pltpu.async_copy(upd_hbm.at[vb_s], upd_v, gsem)
      pltpu.make_async_copy(out_hbm.at[vi_s], acc_v, gsem).wait()
      pltpu.make_async_copy(upd_hbm.at[vb_s], upd_v, gsem).wait()
      add_rows(acc_v, upd_v, KB)
      pltpu.async_copy(acc_v, out_hbm.at[vi_s], ssem).wait()
      return carry

    lax.fori_loop(jnp.int32(0), nbm, main_body, jnp.int32(0))

    nbo = (cnto + (_L - 1)) // _L

    def ovf_body(j, carry):
      base = j * _L
      valid = (base + lanes) < cnto
      vi = oi_v[pl.ds(base, _L)]
      vb = ob_v[pl.ds(base, _L)]
      vim = jnp.where(valid, vi, M + lanes)
      occ1, _ = plsc.scan_count(vim)
      occ = occ1 - 1
      rounds = jnp.max(jnp.where(valid, occ, 0)) + 1

      def round_body(r, rc):
        active = valid & (occ == r)
        f = jnp.broadcast_to(
            plsc.all_reduce_ffs(active), (_L,)).astype(jnp.int32)
        fb = base + f
        vi_o[...] = jnp.where(active, vi, plsc.load_gather(oi_v, [fb]))
        vb_o[...] = jnp.where(active, vb, plsc.load_gather(ob_v, [fb]))
        pltpu.async_copy(out_hbm.at[vi_o], acc_o, gsem)
        pltpu.async_copy(upd_hbm.at[vb_o], upd_o, gsem).wait()
        pltpu.make_async_copy(out_hbm.at[vi_o], acc_o, gsem).wait()
        add_rows(acc_o, upd_o, _L)
        pltpu.async_copy(acc_o, out_hbm.at[vi_o], ssem).wait()
        return rc

      lax.fori_loop(jnp.int32(0), rounds, round_body, jnp.int32(0))
      return carry

    lax.fori_loop(jnp.int32(0), nbo, ovf_body, jnp.int32(0))

  return pl.kernel(
      body,
      out_type=(),
      mesh=mesh,
      compiler_params=pltpu.CompilerParams(needs_layout_passes=False),
      scratch_types=[
          pltpu.VMEM((B,), jnp.int32),
          pltpu.VMEM((B,), jnp.int32),
          pltpu.VMEM((rpw_pad,), jnp.int32),
          pltpu.VMEM((rpw_pad,), jnp.int32),
          pltpu.VMEM((B,), jnp.int32),
          pltpu.VMEM((B,), jnp.int32),
          pltpu.VMEM((rpw_pad,), jnp.int32),
          pltpu.VMEM((KB, D), jnp.float32),
          pltpu.VMEM((KB, D), jnp.float32),
          pltpu.VMEM((KB,), jnp.int32),
          pltpu.VMEM((KB,), jnp.int32),
          pltpu.VMEM((_L, D), jnp.float32),
          pltpu.VMEM((_L, D), jnp.float32),
          pltpu.VMEM((_L,), jnp.int32),
          pltpu.VMEM((_L,), jnp.int32),
          pltpu.SemaphoreType.DMA,
          pltpu.SemaphoreType.DMA,
      ],
  )


def kernel(varRef, indice, updates, mask, axis):
  M, D = varRef.shape
  B = indice.shape[0]
  idx = (indice + axis).astype(jnp.int32)
  msk = jnp.where(mask, jnp.int32(1), jnp.int32(0))
  out_ref = jax.new_ref(varRef)
  _make_scatter_add(M, D, B)(out_ref, idx, msk, updates)
  return out_ref[...]
